# baseline (speedup 1.0000x reference)
import functools

import jax
import jax.numpy as jnp
from jax import lax
from jax.experimental import pallas as pl
from jax.experimental.pallas import tpu as pltpu
from jax.experimental.pallas import tpu_sc as plsc

EMBED = 64
CTX = 20
NSAMP = 64
NQ = EMBED // 16
XOPAD = 32


def _sc_gather_body(XH_hbm, XO_hbm, NH_hbm, NO_hbm, U2_hbm, V2_hbm,
                    sim_hbm, neg_hbm,
                    xh, xo, nh, no, uctx, vctx, nu, nv, simloc, negloc,
                    sem_a, sem_b, *, per):
    c = lax.axis_index("c")
    s = lax.axis_index("s")
    wid = s * 2 + c
    base = wid * per

    pltpu.sync_copy(XH_hbm.at[pl.ds(base, per)], xh)
    pltpu.sync_copy(XO_hbm.at[pl.ds(base, per)], xo)
    pltpu.sync_copy(NH_hbm.at[pl.ds(base, per)], nh)
    pltpu.sync_copy(NO_hbm.at[pl.ds(base, per)], no)

    def issue(buf, sem, e):
        for cp in (
            pltpu.make_async_copy(U2_hbm.at[xh.at[e]], uctx.at[buf], sem),
            pltpu.make_async_copy(V2_hbm.at[xh.at[e]], vctx.at[buf], sem),
            pltpu.make_async_copy(U2_hbm.at[nh.at[e]], nu.at[buf], sem),
            pltpu.make_async_copy(V2_hbm.at[nh.at[e]], nv.at[buf], sem),
        ):
            cp.start()

    def wait(buf, sem):
        pltpu.make_async_copy(U2_hbm.at[xh.at[0]], uctx.at[buf], sem).wait()
        pltpu.make_async_copy(V2_hbm.at[xh.at[0]], vctx.at[buf], sem).wait()
        pltpu.make_async_copy(U2_hbm.at[nh.at[0]], nu.at[buf], sem).wait()
        pltpu.make_async_copy(V2_hbm.at[nh.at[0]], nv.at[buf], sem).wait()

    zero = jnp.zeros((16,), jnp.float32)

    def compute(buf, e):
        uc, vc, nuB, nvB = uctx.at[buf], vctx.at[buf], nu.at[buf], nv.at[buf]
        xov = [xo[e, pl.ds(g * 16, 16)] for g in range(2)]
        sim4 = [zero] * NQ
        for cc in range(CTX):
            off = xov[cc // 16][cc % 16]
            for q in range(NQ):
                sim4[q] = sim4[q] + (uc[cc, pl.ds(off + q * 16, 16)] *
                                     vc[cc, pl.ds(off + q * 16, 16)])
        for q in range(NQ):
            simloc[e, pl.ds(q * 16, 16)] = sim4[q]

        nov = [no[e, pl.ds(g * 16, 16)] for g in range(NQ)]
        offs = [nov[j // 16][j % 16] for j in range(NSAMP)]
        su4 = [zero] * NQ
        for j in range(NSAMP):
            for q in range(NQ):
                su4[q] = su4[q] + nuB[j, pl.ds(offs[j] + q * 16, 16)]
        neg4 = [zero] * NQ
        for j in range(NSAMP):
            w = su4[j // 16][j % 16]
            for q in range(NQ):
                neg4[q] = neg4[q] + w * nvB[j, pl.ds(offs[j] + q * 16, 16)]
        for q in range(NQ):
            negloc[e, pl.ds(q * 16, 16)] = neg4[q]

    issue(0, sem_a, 0)

    @pl.loop(0, per, step=2)
    def _pipe(e):
        issue(1, sem_b, e + 1)
        wait(0, sem_a)
        compute(0, e)
        issue(0, sem_a, jnp.minimum(e + 2, per - 1))
        wait(1, sem_b)
        compute(1, e + 1)

    wait(0, sem_a)

    pltpu.sync_copy(simloc, sim_hbm.at[pl.ds(base, per)])
    pltpu.sync_copy(negloc, neg_hbm.at[pl.ds(base, per)])


def _loss_body(sim_ref, neg_ref, out_ref, *, batch):
    x = sim_ref[...]
    y = -neg_ref[...]

    def log_sigmoid(t):
        return jnp.minimum(t, 0.0) - jnp.log1p(jnp.exp(-jnp.abs(t)))

    total = jnp.sum(log_sigmoid(x)) + jnp.sum(log_sigmoid(y))
    out_ref[0, 0] = -total / float(batch)


def kernel(X, N, neg_samples, batch_size, embed_u, embed_v):
    del N, batch_size
    B = X.shape[0]
    nw = 32
    per = B // nw
    vocab = embed_u.shape[0]

    U2 = embed_u.reshape(vocab // 2, 2 * EMBED)
    V2 = embed_v.reshape(vocab // 2, 2 * EMBED)
    XH = jnp.right_shift(X, 1)
    XO = jnp.pad(jnp.left_shift(jnp.bitwise_and(X, 1), 6),
                 ((0, 0), (0, XOPAD - CTX)))
    NH = jnp.right_shift(neg_samples, 1)
    NO = jnp.left_shift(jnp.bitwise_and(neg_samples, 1), 6)

    mesh = plsc.VectorSubcoreMesh(core_axis_name="c", subcore_axis_name="s")
    sc = pl.kernel(
        functools.partial(_sc_gather_body, per=per),
        out_type=(
            jax.ShapeDtypeStruct((B, EMBED), jnp.float32),
            jax.ShapeDtypeStruct((B, EMBED), jnp.float32),
        ),
        mesh=mesh,
        scratch_types=(
            pltpu.VMEM((per, CTX), jnp.int32),
            pltpu.VMEM((per, XOPAD), jnp.int32),
            pltpu.VMEM((per, NSAMP), jnp.int32),
            pltpu.VMEM((per, NSAMP), jnp.int32),
            pltpu.VMEM((2, CTX, 2 * EMBED), jnp.float32),
            pltpu.VMEM((2, CTX, 2 * EMBED), jnp.float32),
            pltpu.VMEM((2, NSAMP, 2 * EMBED), jnp.float32),
            pltpu.VMEM((2, NSAMP, 2 * EMBED), jnp.float32),
            pltpu.VMEM((per, EMBED), jnp.float32),
            pltpu.VMEM((per, EMBED), jnp.float32),
            pltpu.SemaphoreType.DMA,
            pltpu.SemaphoreType.DMA,
        ),
        compiler_params=pltpu.CompilerParams(use_tc_tiling_on_sc=False),
    )
    sim, neg = sc(XH, XO, NH, NO, U2, V2)

    loss = pl.pallas_call(
        functools.partial(_loss_body, batch=B),
        out_shape=jax.ShapeDtypeStruct((1, 1), jnp.float32),
        out_specs=pl.BlockSpec(memory_space=pltpu.SMEM),
    )(sim, neg)
    return loss[0, 0]


# speedup vs baseline: 1.4909x; 1.4909x over previous
import functools

import jax
import jax.numpy as jnp
from jax import lax
from jax.experimental import pallas as pl
from jax.experimental.pallas import tpu as pltpu
from jax.experimental.pallas import tpu_sc as plsc

EMBED = 64
CTX = 20
CTXP = 24
NSAMP = 64
NQ = EMBED // 16
XOPAD = 32
TR_COLS = 2048
SPLIT = 245 * TR_COLS


def _tr_body(in0_ref, in1_ref, out_ref):
    out_ref[:, 0:EMBED] = in0_ref[...].T
    out_ref[:, EMBED:2 * EMBED] = in1_ref[...].T


def _transpose_table(tt):
    grid = SPLIT // TR_COLS
    last = (tt.shape[1] + TR_COLS - 1) // TR_COLS - 1
    return pl.pallas_call(
        _tr_body,
        grid=(grid,),
        in_specs=[
            pl.BlockSpec((EMBED, TR_COLS), lambda i: (0, i)),
            pl.BlockSpec((EMBED, TR_COLS),
                         lambda i: (0, jnp.minimum(i + grid, last))),
        ],
        out_specs=pl.BlockSpec((TR_COLS, 2 * EMBED), lambda i: (i, 0)),
        out_shape=jax.ShapeDtypeStruct((SPLIT, 2 * EMBED), jnp.float32),
    )(tt, tt)


def _sc_gather_body(XH_hbm, XO_hbm, NH_hbm, NO_hbm, U2_hbm, V2_hbm,
                    sim_hbm, neg_hbm,
                    xh, xo, nh, no, uctx, vctx, nu, nv, simloc, negloc,
                    sem_a, sem_b, *, per):
    c = lax.axis_index("c")
    s = lax.axis_index("s")
    wid = s * 2 + c
    base = wid * per

    pltpu.sync_copy(XH_hbm.at[pl.ds(base * CTXP, per * CTXP)], xh)
    pltpu.sync_copy(XO_hbm.at[pl.ds(base * XOPAD, per * XOPAD)], xo)
    pltpu.sync_copy(NH_hbm.at[pl.ds(base * NSAMP, per * NSAMP)], nh)
    pltpu.sync_copy(NO_hbm.at[pl.ds(base * NSAMP, per * NSAMP)], no)

    def issue(buf, sem, e):
        for cp in (
            pltpu.make_async_copy(
                U2_hbm.at[xh.at[pl.ds(e * CTXP, CTX)]], uctx.at[buf], sem),
            pltpu.make_async_copy(
                V2_hbm.at[xh.at[pl.ds(e * CTXP, CTX)]], vctx.at[buf], sem),
            pltpu.make_async_copy(
                U2_hbm.at[nh.at[pl.ds(e * NSAMP, NSAMP)]], nu.at[buf], sem),
            pltpu.make_async_copy(
                V2_hbm.at[nh.at[pl.ds(e * NSAMP, NSAMP)]], nv.at[buf], sem),
        ):
            cp.start()

    def wait(buf, sem):
        idx0 = xh.at[pl.ds(0, CTX)]
        n0 = nh.at[pl.ds(0, NSAMP)]
        pltpu.make_async_copy(U2_hbm.at[idx0], uctx.at[buf], sem).wait()
        pltpu.make_async_copy(V2_hbm.at[idx0], vctx.at[buf], sem).wait()
        pltpu.make_async_copy(U2_hbm.at[n0], nu.at[buf], sem).wait()
        pltpu.make_async_copy(V2_hbm.at[n0], nv.at[buf], sem).wait()

    zero = jnp.zeros((16,), jnp.float32)

    def compute(buf, e):
        uc, vc, nuB, nvB = uctx.at[buf], vctx.at[buf], nu.at[buf], nv.at[buf]
        xov = [xo[pl.ds(e * XOPAD + g * 16, 16)] for g in range(2)]
        sim4 = [zero] * NQ
        for cc in range(CTX):
            off = xov[cc // 16][cc % 16]
            for q in range(NQ):
                sim4[q] = sim4[q] + (uc[cc, pl.ds(off + q * 16, 16)] *
                                     vc[cc, pl.ds(off + q * 16, 16)])
        for q in range(NQ):
            simloc[pl.ds(e * EMBED + q * 16, 16)] = sim4[q]

        nov = [no[pl.ds(e * NSAMP + g * 16, 16)] for g in range(NQ)]
        offs = [nov[j // 16][j % 16] for j in range(NSAMP)]
        su4 = [zero] * NQ
        for j in range(NSAMP):
            for q in range(NQ):
                su4[q] = su4[q] + nuB[j, pl.ds(offs[j] + q * 16, 16)]
        neg4 = [zero] * NQ
        for j in range(NSAMP):
            w = su4[j // 16][j % 16]
            for q in range(NQ):
                neg4[q] = neg4[q] + w * nvB[j, pl.ds(offs[j] + q * 16, 16)]
        for q in range(NQ):
            negloc[pl.ds(e * EMBED + q * 16, 16)] = neg4[q]

    issue(0, sem_a, 0)

    @pl.loop(0, per, step=2)
    def _pipe(e):
        issue(1, sem_b, e + 1)
        wait(0, sem_a)
        compute(0, e)
        issue(0, sem_a, jnp.minimum(e + 2, per - 1))
        wait(1, sem_b)
        compute(1, e + 1)

    wait(0, sem_a)

    pltpu.sync_copy(simloc, sim_hbm.at[pl.ds(base * EMBED, per * EMBED)])
    pltpu.sync_copy(negloc, neg_hbm.at[pl.ds(base * EMBED, per * EMBED)])


def _loss_body(sim_ref, neg_ref, out_ref, *, batch):
    x = sim_ref[...]
    y = -neg_ref[...]

    def log_sigmoid(t):
        return jnp.minimum(t, 0.0) - jnp.log1p(jnp.exp(-jnp.abs(t)))

    total = jnp.sum(log_sigmoid(x)) + jnp.sum(log_sigmoid(y))
    out_ref[0, 0] = -total / float(batch)


def kernel(X, N, neg_samples, batch_size, embed_u, embed_v):
    del N, batch_size
    B = X.shape[0]
    nw = 32
    per = B // nw
    vocab = embed_u.shape[0]

    U2 = _transpose_table(embed_u.T)
    V2 = _transpose_table(embed_v.T)

    xs = (X >= SPLIT).astype(jnp.int32)
    ns = (neg_samples >= SPLIT).astype(jnp.int32)
    XH = jnp.pad(X - xs * SPLIT, ((0, 0), (0, CTXP - CTX))).reshape(-1)
    XO = jnp.pad(xs * EMBED, ((0, 0), (0, XOPAD - CTX))).reshape(-1)
    NH = (neg_samples - ns * SPLIT).reshape(-1)
    NO = (ns * EMBED).reshape(-1)

    mesh = plsc.VectorSubcoreMesh(core_axis_name="c", subcore_axis_name="s")
    sc = pl.kernel(
        functools.partial(_sc_gather_body, per=per),
        out_type=(
            jax.ShapeDtypeStruct((B * EMBED,), jnp.float32),
            jax.ShapeDtypeStruct((B * EMBED,), jnp.float32),
        ),
        mesh=mesh,
        scratch_types=(
            pltpu.VMEM((per * CTXP,), jnp.int32),
            pltpu.VMEM((per * XOPAD,), jnp.int32),
            pltpu.VMEM((per * NSAMP,), jnp.int32),
            pltpu.VMEM((per * NSAMP,), jnp.int32),
            pltpu.VMEM((2, CTX, 2 * EMBED), jnp.float32),
            pltpu.VMEM((2, CTX, 2 * EMBED), jnp.float32),
            pltpu.VMEM((2, NSAMP, 2 * EMBED), jnp.float32),
            pltpu.VMEM((2, NSAMP, 2 * EMBED), jnp.float32),
            pltpu.VMEM((per * EMBED,), jnp.float32),
            pltpu.VMEM((per * EMBED,), jnp.float32),
            pltpu.SemaphoreType.DMA,
            pltpu.SemaphoreType.DMA,
        ),
    )
    sim, neg = sc(XH, XO, NH, NO, U2, V2)

    loss = pl.pallas_call(
        functools.partial(_loss_body, batch=B),
        out_shape=jax.ShapeDtypeStruct((1, 1), jnp.float32),
        out_specs=pl.BlockSpec(memory_space=pltpu.SMEM),
    )(sim.reshape(B * EMBED // 128, 128), neg.reshape(B * EMBED // 128, 128))
    return loss[0, 0]


# speedup vs baseline: 1.8223x; 1.2223x over previous
import functools

import jax
import jax.numpy as jnp
from jax import lax
from jax.experimental import pallas as pl
from jax.experimental.pallas import tpu as pltpu
from jax.experimental.pallas import tpu_sc as plsc

EMBED = 64
CTX = 20
CTXP = 24
NSAMP = 64
NQ = EMBED // 16
TR_COLS = 4096


def _tr_body(u_ref, v_ref, out_ref):
    eye = jnp.eye(EMBED, dtype=jnp.float32)
    out_ref[:, 0:EMBED] = lax.dot_general(
        u_ref[...], eye, (((0,), (0,)), ((), ())),
        preferred_element_type=jnp.float32)
    out_ref[:, EMBED:2 * EMBED] = lax.dot_general(
        v_ref[...], eye, (((0,), (0,)), ((), ())),
        preferred_element_type=jnp.float32)


def _combine_tables(ut, vt):
    v = ut.shape[1]
    grid = (v + TR_COLS - 1) // TR_COLS
    return pl.pallas_call(
        _tr_body,
        grid=(grid,),
        in_specs=[
            pl.BlockSpec((EMBED, TR_COLS), lambda i: (0, i)),
            pl.BlockSpec((EMBED, TR_COLS), lambda i: (0, i)),
        ],
        out_specs=pl.BlockSpec((TR_COLS, 2 * EMBED), lambda i: (i, 0)),
        out_shape=jax.ShapeDtypeStruct((v, 2 * EMBED), jnp.float32),
    )(ut, vt)


def _sc_gather_body(XP_hbm, NP_hbm, W_hbm, sim_hbm, neg_hbm,
                    xp, np_, wctx, wneg, simloc, negloc,
                    sem_a, sem_b, *, per):
    c = lax.axis_index("c")
    s = lax.axis_index("s")
    wid = s * 2 + c
    base = wid * per

    pltpu.sync_copy(XP_hbm.at[pl.ds(base * CTXP, per * CTXP)], xp)
    pltpu.sync_copy(NP_hbm.at[pl.ds(base * NSAMP, per * NSAMP)], np_)

    def issue(buf, sem, e):
        pltpu.make_async_copy(
            W_hbm.at[xp.at[pl.ds(e * CTXP, CTX)]], wctx.at[buf], sem).start()
        pltpu.make_async_copy(
            W_hbm.at[np_.at[pl.ds(e * NSAMP, NSAMP)]], wneg.at[buf],
            sem).start()

    def wait(buf, sem):
        pltpu.make_async_copy(
            W_hbm.at[xp.at[pl.ds(0, CTX)]], wctx.at[buf], sem).wait()
        pltpu.make_async_copy(
            W_hbm.at[np_.at[pl.ds(0, NSAMP)]], wneg.at[buf], sem).wait()

    zero = jnp.zeros((16,), jnp.float32)

    def compute(buf, e):
        wc, wn = wctx.at[buf], wneg.at[buf]
        sim4 = [zero] * NQ
        for cc in range(CTX):
            for q in range(NQ):
                sim4[q] = sim4[q] + (wc[cc, pl.ds(q * 16, 16)] *
                                     wc[cc, pl.ds(EMBED + q * 16, 16)])
        for q in range(NQ):
            simloc[pl.ds(e * EMBED + q * 16, 16)] = sim4[q]

        su4 = [zero] * NQ
        for j in range(NSAMP):
            for q in range(NQ):
                su4[q] = su4[q] + wn[j, pl.ds(q * 16, 16)]
        neg4 = [zero] * NQ
        for j in range(NSAMP):
            w = su4[j // 16][j % 16]
            for q in range(NQ):
                neg4[q] = neg4[q] + w * wn[j, pl.ds(EMBED + q * 16, 16)]
        for q in range(NQ):
            negloc[pl.ds(e * EMBED + q * 16, 16)] = neg4[q]

    issue(0, sem_a, 0)

    @pl.loop(0, per, step=2)
    def _pipe(e):
        issue(1, sem_b, e + 1)
        wait(0, sem_a)
        compute(0, e)
        issue(0, sem_a, jnp.minimum(e + 2, per - 1))
        wait(1, sem_b)
        compute(1, e + 1)

    wait(0, sem_a)

    pltpu.sync_copy(simloc, sim_hbm.at[pl.ds(base * EMBED, per * EMBED)])
    pltpu.sync_copy(negloc, neg_hbm.at[pl.ds(base * EMBED, per * EMBED)])


def _loss_body(sim_ref, neg_ref, out_ref, *, batch):
    x = sim_ref[...]
    y = -neg_ref[...]

    def log_sigmoid(t):
        return jnp.minimum(t, 0.0) - jnp.log1p(jnp.exp(-jnp.abs(t)))

    total = jnp.sum(log_sigmoid(x)) + jnp.sum(log_sigmoid(y))
    out_ref[0, 0] = -total / float(batch)


def kernel(X, N, neg_samples, batch_size, embed_u, embed_v):
    del N, batch_size
    B = X.shape[0]
    nw = 32
    per = B // nw

    W = _combine_tables(embed_u.T, embed_v.T)

    XP = jnp.pad(X, ((0, 0), (0, CTXP - CTX))).reshape(-1)
    NP = neg_samples.reshape(-1)

    mesh = plsc.VectorSubcoreMesh(core_axis_name="c", subcore_axis_name="s")
    sc = pl.kernel(
        functools.partial(_sc_gather_body, per=per),
        out_type=(
            jax.ShapeDtypeStruct((B * EMBED,), jnp.float32),
            jax.ShapeDtypeStruct((B * EMBED,), jnp.float32),
        ),
        mesh=mesh,
        scratch_types=(
            pltpu.VMEM((per * CTXP,), jnp.int32),
            pltpu.VMEM((per * NSAMP,), jnp.int32),
            pltpu.VMEM((2, CTX, 2 * EMBED), jnp.float32),
            pltpu.VMEM((2, NSAMP, 2 * EMBED), jnp.float32),
            pltpu.VMEM((per * EMBED,), jnp.float32),
            pltpu.VMEM((per * EMBED,), jnp.float32),
            pltpu.SemaphoreType.DMA,
            pltpu.SemaphoreType.DMA,
        ),
    )
    sim, neg = sc(XP, NP, W)

    loss = pl.pallas_call(
        functools.partial(_loss_body, batch=B),
        out_shape=jax.ShapeDtypeStruct((1, 1), jnp.float32),
        out_specs=pl.BlockSpec(memory_space=pltpu.SMEM),
    )(sim.reshape(B * EMBED // 128, 128), neg.reshape(B * EMBED // 128, 128))
    return loss[0, 0]


# speedup vs baseline: 1.9601x; 1.0756x over previous
import functools

import jax
import jax.numpy as jnp
from jax import lax
from jax.experimental import pallas as pl
from jax.experimental.pallas import tpu as pltpu
from jax.experimental.pallas import tpu_sc as plsc

EMBED = 64
CTX = 20
CTXP = 24
NSAMP = 64
NQ = EMBED // 16
TR_COLS = 8192


def _tr_body(u_ref, v_ref, out_ref):
    eye = jnp.eye(EMBED, dtype=jnp.float32)
    out_ref[:, 0:EMBED] = lax.dot_general(
        u_ref[...], eye, (((0,), (0,)), ((), ())),
        preferred_element_type=jnp.float32)
    out_ref[:, EMBED:2 * EMBED] = lax.dot_general(
        v_ref[...], eye, (((0,), (0,)), ((), ())),
        preferred_element_type=jnp.float32)


def _combine_tables(ut, vt):
    v = ut.shape[1]
    grid = (v + TR_COLS - 1) // TR_COLS
    return pl.pallas_call(
        _tr_body,
        grid=(grid,),
        in_specs=[
            pl.BlockSpec((EMBED, TR_COLS), lambda i: (0, i)),
            pl.BlockSpec((EMBED, TR_COLS), lambda i: (0, i)),
        ],
        out_specs=pl.BlockSpec((TR_COLS, 2 * EMBED), lambda i: (i, 0)),
        out_shape=jax.ShapeDtypeStruct((v, 2 * EMBED), jnp.float32),
    )(ut, vt)


def _sc_gather_body(XP_hbm, NP_hbm, W_hbm, sim_hbm, neg_hbm,
                    xp, np_, wctx, wneg, simloc, negloc,
                    sem_a, sem_b, *, per):
    c = lax.axis_index("c")
    s = lax.axis_index("s")
    wid = s * 2 + c
    base = wid * per

    pltpu.sync_copy(XP_hbm.at[pl.ds(base * CTXP, per * CTXP)], xp)
    pltpu.sync_copy(NP_hbm.at[pl.ds(base * NSAMP, per * NSAMP)], np_)

    def issue(buf, sem, e):
        pltpu.make_async_copy(
            W_hbm.at[xp.at[pl.ds(e * CTXP, CTX)]], wctx.at[buf], sem).start()
        pltpu.make_async_copy(
            W_hbm.at[np_.at[pl.ds(e * NSAMP, NSAMP)]], wneg.at[buf],
            sem).start()

    def wait(buf, sem):
        pltpu.make_async_copy(
            W_hbm.at[xp.at[pl.ds(0, CTX)]], wctx.at[buf], sem).wait()
        pltpu.make_async_copy(
            W_hbm.at[np_.at[pl.ds(0, NSAMP)]], wneg.at[buf], sem).wait()

    zero = jnp.zeros((16,), jnp.float32)

    def compute(buf, e):
        LANES = 4
        wc, wn = wctx.at[buf], wneg.at[buf]
        sim4 = [[zero] * NQ for _ in range(LANES)]
        for cc in range(CTX):
            a = sim4[cc % LANES]
            for q in range(NQ):
                a[q] = a[q] + (wc[cc, pl.ds(q * 16, 16)] *
                               wc[cc, pl.ds(EMBED + q * 16, 16)])
        for q in range(NQ):
            simloc[pl.ds(e * EMBED + q * 16, 16)] = (
                (sim4[0][q] + sim4[1][q]) + (sim4[2][q] + sim4[3][q]))

        sup = [[zero] * NQ for _ in range(LANES)]
        for j in range(NSAMP):
            a = sup[j % LANES]
            for q in range(NQ):
                a[q] = a[q] + wn[j, pl.ds(q * 16, 16)]
        su4 = [(sup[0][q] + sup[1][q]) + (sup[2][q] + sup[3][q])
               for q in range(NQ)]
        negp = [[zero] * NQ for _ in range(LANES)]
        for j in range(NSAMP):
            w = su4[j // 16][j % 16]
            a = negp[j % LANES]
            for q in range(NQ):
                a[q] = a[q] + w * wn[j, pl.ds(EMBED + q * 16, 16)]
        for q in range(NQ):
            negloc[pl.ds(e * EMBED + q * 16, 16)] = (
                (negp[0][q] + negp[1][q]) + (negp[2][q] + negp[3][q]))

    issue(0, sem_a, 0)

    @pl.loop(0, per, step=2)
    def _pipe(e):
        issue(1, sem_b, e + 1)
        wait(0, sem_a)
        compute(0, e)
        issue(0, sem_a, jnp.minimum(e + 2, per - 1))
        wait(1, sem_b)
        compute(1, e + 1)

    wait(0, sem_a)

    pltpu.sync_copy(simloc, sim_hbm.at[pl.ds(base * EMBED, per * EMBED)])
    pltpu.sync_copy(negloc, neg_hbm.at[pl.ds(base * EMBED, per * EMBED)])


def _loss_body(sim_ref, neg_ref, out_ref, *, batch):
    x = sim_ref[...]
    y = -neg_ref[...]

    def log_sigmoid(t):
        return jnp.minimum(t, 0.0) - jnp.log1p(jnp.exp(-jnp.abs(t)))

    total = jnp.sum(log_sigmoid(x)) + jnp.sum(log_sigmoid(y))
    out_ref[0, 0] = -total / float(batch)


def kernel(X, N, neg_samples, batch_size, embed_u, embed_v):
    del N, batch_size
    B = X.shape[0]
    nw = 32
    per = B // nw

    W = _combine_tables(embed_u.T, embed_v.T)

    XP = jnp.pad(X, ((0, 0), (0, CTXP - CTX))).reshape(-1)
    NP = neg_samples.reshape(-1)

    mesh = plsc.VectorSubcoreMesh(core_axis_name="c", subcore_axis_name="s")
    sc = pl.kernel(
        functools.partial(_sc_gather_body, per=per),
        out_type=(
            jax.ShapeDtypeStruct((B * EMBED,), jnp.float32),
            jax.ShapeDtypeStruct((B * EMBED,), jnp.float32),
        ),
        mesh=mesh,
        scratch_types=(
            pltpu.VMEM((per * CTXP,), jnp.int32),
            pltpu.VMEM((per * NSAMP,), jnp.int32),
            pltpu.VMEM((2, CTX, 2 * EMBED), jnp.float32),
            pltpu.VMEM((2, NSAMP, 2 * EMBED), jnp.float32),
            pltpu.VMEM((per * EMBED,), jnp.float32),
            pltpu.VMEM((per * EMBED,), jnp.float32),
            pltpu.SemaphoreType.DMA,
            pltpu.SemaphoreType.DMA,
        ),
    )
    sim, neg = sc(XP, NP, W)

    loss = pl.pallas_call(
        functools.partial(_loss_body, batch=B),
        out_shape=jax.ShapeDtypeStruct((1, 1), jnp.float32),
        out_specs=pl.BlockSpec(memory_space=pltpu.SMEM),
    )(sim.reshape(B * EMBED // 128, 128), neg.reshape(B * EMBED // 128, 128))
    return loss[0, 0]


# speedup vs baseline: 2.0524x; 1.0471x over previous
import functools

import jax
import jax.numpy as jnp
from jax import lax
from jax.experimental import pallas as pl
from jax.experimental.pallas import tpu as pltpu
from jax.experimental.pallas import tpu_sc as plsc

EMBED = 64
CTX = 20
CTXP = 24
NSAMP = 64
NQ = EMBED // 16
TR_COLS = 16384
NBUF = 4


def _tr_body(u_ref, v_ref, out_ref):
    eye = jnp.eye(EMBED, dtype=jnp.float32)
    out_ref[:, 0:EMBED] = lax.dot_general(
        u_ref[...], eye, (((0,), (0,)), ((), ())),
        preferred_element_type=jnp.float32)
    out_ref[:, EMBED:2 * EMBED] = lax.dot_general(
        v_ref[...], eye, (((0,), (0,)), ((), ())),
        preferred_element_type=jnp.float32)


def _combine_tables(ut, vt):
    v = ut.shape[1]
    grid = (v + TR_COLS - 1) // TR_COLS
    return pl.pallas_call(
        _tr_body,
        grid=(grid,),
        in_specs=[
            pl.BlockSpec((EMBED, TR_COLS), lambda i: (0, i)),
            pl.BlockSpec((EMBED, TR_COLS), lambda i: (0, i)),
        ],
        out_specs=pl.BlockSpec((TR_COLS, 2 * EMBED), lambda i: (i, 0)),
        out_shape=jax.ShapeDtypeStruct((v, 2 * EMBED), jnp.float32),
    )(ut, vt)


def _sc_gather_body(XP_hbm, NP_hbm, W_hbm, sim_hbm, neg_hbm,
                    xp, np_, wctx, wneg, simloc, negloc, sems, *, per):
    c = lax.axis_index("c")
    s = lax.axis_index("s")
    wid = s * 2 + c
    base = wid * per

    pltpu.sync_copy(XP_hbm.at[pl.ds(base * CTXP, per * CTXP)], xp)
    pltpu.sync_copy(NP_hbm.at[pl.ds(base * NSAMP, per * NSAMP)], np_)

    def issue(buf, e):
        pltpu.make_async_copy(
            W_hbm.at[xp.at[pl.ds(e * CTXP, CTX)]], wctx.at[buf],
            sems.at[buf]).start()
        pltpu.make_async_copy(
            W_hbm.at[np_.at[pl.ds(e * NSAMP, NSAMP)]], wneg.at[buf],
            sems.at[buf]).start()

    def wait(buf):
        pltpu.make_async_copy(
            W_hbm.at[xp.at[pl.ds(0, CTX)]], wctx.at[buf],
            sems.at[buf]).wait()
        pltpu.make_async_copy(
            W_hbm.at[np_.at[pl.ds(0, NSAMP)]], wneg.at[buf],
            sems.at[buf]).wait()

    zero = jnp.zeros((16,), jnp.float32)

    def compute(buf, e):
        wc, wn = wctx.at[buf], wneg.at[buf]
        sim4 = [zero] * NQ
        for cc in range(CTX):
            for q in range(NQ):
                sim4[q] = sim4[q] + (wc[cc, pl.ds(q * 16, 16)] *
                                     wc[cc, pl.ds(EMBED + q * 16, 16)])
        for q in range(NQ):
            simloc[pl.ds(e * EMBED + q * 16, 16)] = sim4[q]

        su4 = [zero] * NQ
        for j in range(NSAMP):
            for q in range(NQ):
                su4[q] = su4[q] + wn[j, pl.ds(q * 16, 16)]
        neg4 = [zero] * NQ
        for j in range(NSAMP):
            w = su4[j // 16][j % 16]
            for q in range(NQ):
                neg4[q] = neg4[q] + w * wn[j, pl.ds(EMBED + q * 16, 16)]
        for q in range(NQ):
            negloc[pl.ds(e * EMBED + q * 16, 16)] = neg4[q]

    for b in range(NBUF):
        issue(b, b)

    @pl.loop(0, per, step=NBUF)
    def _pipe(e):
        for b in range(NBUF):
            wait(b)
            compute(b, e + b)
            issue(b, jnp.minimum(e + b + NBUF, per - 1))

    for b in range(NBUF):
        wait(b)

    pltpu.sync_copy(simloc, sim_hbm.at[pl.ds(base * EMBED, per * EMBED)])
    pltpu.sync_copy(negloc, neg_hbm.at[pl.ds(base * EMBED, per * EMBED)])


def _loss_body(sim_ref, neg_ref, out_ref, *, batch):
    x = sim_ref[...]
    y = -neg_ref[...]

    def log_sigmoid(t):
        return jnp.minimum(t, 0.0) - jnp.log1p(jnp.exp(-jnp.abs(t)))

    total = jnp.sum(log_sigmoid(x)) + jnp.sum(log_sigmoid(y))
    out_ref[0, 0] = -total / float(batch)


def kernel(X, N, neg_samples, batch_size, embed_u, embed_v):
    del N, batch_size
    B = X.shape[0]
    nw = 32
    per = B // nw

    W = _combine_tables(embed_u.T, embed_v.T)

    XP = jnp.pad(X, ((0, 0), (0, CTXP - CTX))).reshape(-1)
    NP = neg_samples.reshape(-1)

    mesh = plsc.VectorSubcoreMesh(core_axis_name="c", subcore_axis_name="s")
    sc = pl.kernel(
        functools.partial(_sc_gather_body, per=per),
        out_type=(
            jax.ShapeDtypeStruct((B * EMBED,), jnp.float32),
            jax.ShapeDtypeStruct((B * EMBED,), jnp.float32),
        ),
        mesh=mesh,
        scratch_types=(
            pltpu.VMEM((per * CTXP,), jnp.int32),
            pltpu.VMEM((per * NSAMP,), jnp.int32),
            pltpu.VMEM((NBUF, CTX, 2 * EMBED), jnp.float32),
            pltpu.VMEM((NBUF, NSAMP, 2 * EMBED), jnp.float32),
            pltpu.VMEM((per * EMBED,), jnp.float32),
            pltpu.VMEM((per * EMBED,), jnp.float32),
            pltpu.SemaphoreType.DMA((NBUF,)),
        ),
    )
    sim, neg = sc(XP, NP, W)

    loss = pl.pallas_call(
        functools.partial(_loss_body, batch=B),
        out_shape=jax.ShapeDtypeStruct((1, 1), jnp.float32),
        out_specs=pl.BlockSpec(memory_space=pltpu.SMEM),
    )(sim.reshape(B * EMBED // 128, 128), neg.reshape(B * EMBED // 128, 128))
    return loss[0, 0]
